# Initial kernel scaffold; baseline (speedup 1.0000x reference)
import jax
import jax.numpy as jnp
from jax.experimental import pallas as pl


def kernel(x, edge_index, edge_attr, batch, batch_size, W1, a_src1, a_dst1, We1, a_e1, b1, W2, a_src2, a_dst2, We2, a_e2, b2, empty_emb, Rw1, Rb1, Rw2, Rb2):
    raise NotImplementedError("write your pallas kernel here")


# speedup vs baseline: 1.6292x; 1.6292x over previous
import jax
import jax.numpy as jnp
from jax.experimental import pallas as pl


def _gat(x, src, dst, edge_attr, W, a_src, a_dst, We, a_e, b, n_nodes):
    h = x @ W
    alpha_src = jnp.sum(h * a_src, axis=-1)
    alpha_dst = jnp.sum(h * a_dst, axis=-1)
    alpha_e = edge_attr @ (We @ a_e)
    alpha = alpha_src[src] + alpha_dst[dst] + alpha_e
    alpha = jax.nn.leaky_relu(alpha, negative_slope=0.2)
    ex = jnp.exp(alpha)
    denom = jax.ops.segment_sum(ex, dst, num_segments=n_nodes)
    acc = jax.ops.segment_sum(h[src] * ex[:, None], dst, num_segments=n_nodes)
    return acc / (denom + 1e-16)[:, None] + b


def _readout_kernel(pooled_ref, emb_ref, rw1_ref, rb1_ref, rw2_ref, rb2_ref, o_ref):
    pooled = pooled_ref[...]
    mask = jnp.sum(pooled, axis=-1) == 0.0
    pooled = jnp.where(mask[:, None], emb_ref[...][None, :], pooled)
    hid = jnp.maximum(pooled @ rw1_ref[...] + rb1_ref[...][None, :], 0.0)
    o_ref[...] = hid @ rw2_ref[...] + rb2_ref[...][None, :]


def kernel(x, edge_index, edge_attr, batch, batch_size, W1, a_src1, a_dst1,
           We1, a_e1, b1, W2, a_src2, a_dst2, We2, a_e2, b2, empty_emb,
           Rw1, Rb1, Rw2, Rb2):
    src = edge_index[0]
    dst = edge_index[1]
    n = x.shape[0]
    G = 64
    h = jax.nn.relu(_gat(x, src, dst, edge_attr, W1, a_src1, a_dst1, We1, a_e1, b1, n))
    h = _gat(h, src, dst, edge_attr, W2, a_src2, a_dst2, We2, a_e2, b2, n)
    pooled = jax.ops.segment_sum(h, batch, num_segments=G)
    out = pl.pallas_call(
        _readout_kernel,
        out_shape=jax.ShapeDtypeStruct((G, 1), jnp.float32),
    )(pooled, empty_emb, Rw1, Rb1, Rw2, Rb2)
    return out.squeeze(-1)


# speedup vs baseline: 11.9086x; 7.3094x over previous
import dataclasses
import functools

import jax
import jax.numpy as jnp
from jax import lax
from jax.experimental import pallas as pl
from jax.experimental.pallas import tpu as pltpu
from jax.experimental.pallas import tpu_sc as plsc

N = 10000
E = 320000
D = 128
H = 128
G = 64
RH = 256

NC = 2
NS = 16
NW = NC * NS
CW = 128
CB = 80
EPT_P = CB * CW
E_PAD = NW * EPT_P
SB = 8
NP_ = 10240
NB = 1024
EB = 8000
RPTP = NP_ // NS


def _dot(a, b):
    return jnp.dot(a, b, preferred_element_type=jnp.float32)


def _proj(h, av):
    s0 = jnp.sum(h * av[:, 0][None, :], axis=1, keepdims=True)
    s1 = jnp.sum(h * av[:, 1][None, :], axis=1, keepdims=True)
    return jnp.concatenate([s0, s1], axis=1)


def _node_body(x_ref, w_ref, av_ref, h_ref, sad_ref):
    h = _dot(x_ref[...], w_ref[...])
    h_ref[...] = h
    sad_ref[...] = _proj(h, av_ref[...])


def _node_kernel(x, W, av):
    return pl.pallas_call(
        _node_body,
        grid=(NP_ // NB,),
        in_specs=[
            pl.BlockSpec((NB, D), lambda i: (i, 0)),
            pl.BlockSpec((D, H), lambda i: (0, 0)),
            pl.BlockSpec((H, 2), lambda i: (0, 0)),
        ],
        out_specs=[
            pl.BlockSpec((NB, H), lambda i: (i, 0)),
            pl.BlockSpec((NB, 2), lambda i: (i, 0)),
        ],
        out_shape=[
            jax.ShapeDtypeStruct((NP_, H), jnp.float32),
            jax.ShapeDtypeStruct((NP_, 2), jnp.float32),
        ],
    )(x, W, av)


def _edge_body(ea_ref, we1_ref, we2_ref, aev_ref, o_ref):
    ea = ea_ref[...]
    aev = aev_ref[...]
    e1 = _dot(ea, we1_ref[...])
    e2 = _dot(ea, we2_ref[...])
    o1 = jnp.sum(e1 * aev[0][None, :], axis=1, keepdims=True)
    o2 = jnp.sum(e2 * aev[1][None, :], axis=1, keepdims=True)
    o_ref[...] = jnp.concatenate([o1, o2], axis=1)


def _edge_kernel(edge_attr, We1, We2, aev):
    return pl.pallas_call(
        _edge_body,
        grid=(E // EB,),
        in_specs=[
            pl.BlockSpec((EB, 16), lambda i: (i, 0)),
            pl.BlockSpec((16, H), lambda i: (0, 0)),
            pl.BlockSpec((16, H), lambda i: (0, 0)),
            pl.BlockSpec((2, H), lambda i: (0, 0)),
        ],
        out_specs=pl.BlockSpec((EB, 2), lambda i: (i, 0)),
        out_shape=jax.ShapeDtypeStruct((E, 2), jnp.float32),
    )(edge_attr, We1, We2, aev)


def _mid_body(acc_ref, den_ref, b_ref, w_ref, av_ref, h2_ref, sad_ref):
    i = pl.program_id(0)
    acc = acc_ref[0] + acc_ref[1]
    den = jnp.sum(den_ref[:, pl.ds(i * NB, NB)], axis=0)
    h1 = jnp.maximum(acc / (den[:, None] + 1e-16) + b_ref[...], 0.0)
    h2 = _dot(h1, w_ref[...])
    h2_ref[...] = h2
    sad_ref[...] = _proj(h2, av_ref[...])


def _mid_kernel(acc, den, b, W, av):
    return pl.pallas_call(
        _mid_body,
        grid=(NP_ // NB,),
        in_specs=[
            pl.BlockSpec((NC, NB, H), lambda i: (0, i, 0)),
            pl.BlockSpec((NW, NP_), lambda i: (0, 0)),
            pl.BlockSpec((1, H), lambda i: (0, 0)),
            pl.BlockSpec((H, H), lambda i: (0, 0)),
            pl.BlockSpec((H, 2), lambda i: (0, 0)),
        ],
        out_specs=[
            pl.BlockSpec((NB, H), lambda i: (i, 0)),
            pl.BlockSpec((NB, 2), lambda i: (i, 0)),
        ],
        out_shape=[
            jax.ShapeDtypeStruct((NP_, H), jnp.float32),
            jax.ShapeDtypeStruct((NP_, 2), jnp.float32),
        ],
    )(acc, den, b, W, av)


def _final_body(acc_ref, den_ref, batch_ref, b2_ref, emb_ref, rw1_ref,
                rb1_ref, rw2_ref, rb2_ref, o_ref):
    pooled = jnp.zeros((G, H), jnp.float32)
    for blk in range(NP_ // NB):
        sl = pl.ds(blk * NB, NB)
        acc = acc_ref[0, sl, :] + acc_ref[1, sl, :]
        den = jnp.sum(den_ref[:, sl], axis=0)[:, None]
        h2 = acc / (den + 1e-16) + b2_ref[...]
        brow = batch_ref[blk, :][None, :]
        oh = (lax.broadcasted_iota(jnp.int32, (G, NB), 0) == brow)
        pooled = pooled + jnp.dot(
            oh.astype(jnp.float32), h2,
            preferred_element_type=jnp.float32,
            precision=lax.Precision.HIGHEST)
    mask = jnp.sum(pooled, axis=1, keepdims=True) == 0.0
    pooled = jnp.where(mask, emb_ref[...], pooled)
    hid = jnp.maximum(_dot(pooled, rw1_ref[...]) + rb1_ref[...], 0.0)
    o_ref[...] = _dot(hid, rw2_ref[...]) + rb2_ref[...]


def _final_kernel(acc, den, batch2, b2, emb, Rw1, Rb1, Rw2, Rb2):
    return pl.pallas_call(
        _final_body,
        out_shape=jax.ShapeDtypeStruct((G, 1), jnp.float32),
    )(acc, den, batch2, b2, emb, Rw1, Rb1, Rw2, Rb2)


def _sc_compiler_params():
    cp = pltpu.CompilerParams()
    if "needs_layout_passes" in pltpu.CompilerParams.__dataclass_fields__:
        cp = dataclasses.replace(cp, needs_layout_passes=False)
    return cp


def _sc_alpha_pass(src3, dst3, ae3, asrc, adst):
    mesh = plsc.VectorSubcoreMesh(core_axis_name="c", subcore_axis_name="s")

    @functools.partial(
        pl.kernel,
        mesh=mesh,
        compiler_params=_sc_compiler_params(),
        out_type=[
            jax.ShapeDtypeStruct((NW, CB, CW), jnp.float32),
            jax.ShapeDtypeStruct((NW, NP_), jnp.float32),
        ],
        scratch_types=[
            pltpu.VMEM((NP_,), jnp.float32),
            pltpu.VMEM((NP_,), jnp.float32),
            pltpu.VMEM((NP_,), jnp.float32),
            pltpu.VMEM((CB, CW), jnp.int32),
            pltpu.VMEM((CB, CW), jnp.int32),
            pltpu.VMEM((CB, CW), jnp.float32),
            pltpu.VMEM((CB, CW), jnp.float32),
            pltpu.VMEM((16,), jnp.int32),
            pltpu.VMEM((16,), jnp.float32),
        ],
    )
    def k(src_hbm, dst_hbm, ae_hbm, asrc_hbm, adst_hbm, ex_hbm, den_hbm,
          asrc_v, adst_v, den_v, src_v, dst_v, ae_v, ex_v, kbuf, sbuf):
        cid = lax.axis_index("c")
        sid = lax.axis_index("s")
        wid = sid * NC + cid
        iota16 = lax.iota(jnp.int32, 16)

        pltpu.sync_copy(src_hbm.at[wid], src_v)
        pltpu.sync_copy(dst_hbm.at[wid], dst_v)
        pltpu.sync_copy(ae_hbm.at[wid], ae_v)
        pltpu.sync_copy(asrc_hbm, asrc_v)
        pltpu.sync_copy(adst_hbm, adst_v)

        @pl.loop(0, NP_, step=16)
        def _(i):
            den_v[pl.ds(i, 16)] = jnp.zeros((16,), jnp.float32)

        @pl.loop(0, CB)
        def _(r):
            @pl.loop(0, CW, step=16)
            def _(j):
                s16 = src_v[r, pl.ds(j, 16)]
                d16 = dst_v[r, pl.ds(j, 16)]
                a = (plsc.load_gather(asrc_v, [s16])
                     + plsc.load_gather(adst_v, [d16])
                     + ae_v[r, pl.ds(j, 16)])
                a = jnp.where(a >= 0.0, a, 0.2 * a)
                exv = jnp.exp(a)
                ex_v[r, pl.ds(j, 16)] = exv

                ks, vs = plsc.sort_key_val(d16, exv)
                kbuf[...] = ks
                run = vs
                for dlt in (1, 2, 4, 8):
                    sbuf[...] = run
                    idx = jnp.maximum(iota16 - dlt, 0)
                    kd = plsc.load_gather(kbuf, [idx])
                    sd = plsc.load_gather(sbuf, [idx])
                    take = (kd == ks) & (iota16 >= dlt)
                    run = run + jnp.where(take, sd, 0.0)
                knext = plsc.load_gather(kbuf, [jnp.minimum(iota16 + 1, 15)])
                last = (knext != ks) | (iota16 == 15)
                plsc.addupdate_scatter(den_v, [ks], run, mask=last)

        pltpu.sync_copy(ex_v, ex_hbm.at[wid])
        pltpu.sync_copy(den_v, den_hbm.at[wid])

    return k(src3, dst3, ae3, asrc, adst)


def _sc_scatter_pass(h, src3, dst3, ex3):
    mesh = plsc.VectorSubcoreMesh(core_axis_name="c", subcore_axis_name="s")

    @functools.partial(
        pl.kernel,
        mesh=mesh,
        compiler_params=_sc_compiler_params(),
        out_type=jax.ShapeDtypeStruct((NC, NP_, H), jnp.float32),
        scratch_types=[
            pltpu.VMEM((SB, CW), jnp.int32),
            pltpu.VMEM((SB, CW), jnp.int32),
            pltpu.VMEM((SB, CW), jnp.float32),
            pltpu.VMEM((CW, H), jnp.float32),
            pltpu.VMEM_SHARED((NP_, H), jnp.float32),
            pltpu.SemaphoreType.DMA,
        ],
    )
    def k(h_hbm, src_hbm, dst_hbm, ex_hbm, acc_hbm,
          sblk, dblk, eblk, rows, acc_sh, sem_g):
        cid = lax.axis_index("c")
        sid = lax.axis_index("s")
        wid = sid * NC + cid

        @pl.loop(0, CW)
        def _(r):
            @pl.loop(0, H, step=16)
            def _(c2):
                rows[r, pl.ds(c2, 16)] = jnp.zeros((16,), jnp.float32)

        zbase = RPTP * sid
        for j in range(RPTP // CW):
            pltpu.sync_copy(rows, acc_sh.at[pl.ds(zbase + j * CW, CW)])

        plsc.subcore_barrier()

        @pl.loop(0, CB // SB)
        def _(b):
            pltpu.sync_copy(src_hbm.at[wid, pl.ds(b * SB, SB)], sblk)
            pltpu.sync_copy(dst_hbm.at[wid, pl.ds(b * SB, SB)], dblk)
            pltpu.sync_copy(ex_hbm.at[wid, pl.ds(b * SB, SB)], eblk)

            @pl.loop(0, SB)
            def _(i):
                gcopy = pltpu.async_copy(h_hbm.at[sblk.at[i]], rows, sem_g)
                gcopy.wait()
                @pl.loop(0, CW, step=16)
                def _(j):
                    exv = eblk[i, pl.ds(j, 16)]
                    for t in range(16):
                        scv = jnp.full((16,), exv[t], jnp.float32)
                        for v in range(H // 16):
                            rows[j + t, pl.ds(v * 16, 16)] = (
                                rows[j + t, pl.ds(v * 16, 16)] * scv)
                pltpu.sync_copy(rows, acc_sh.at[dblk.at[i]], add=True)

        plsc.subcore_barrier()

        for j in range(RPTP // CW):
            pltpu.sync_copy(acc_sh.at[pl.ds(zbase + j * CW, CW)],
                            acc_hbm.at[cid, pl.ds(zbase + j * CW, CW)])

    return k(h, src3, dst3, ex3)


def kernel(x, edge_index, edge_attr, batch, batch_size, W1, a_src1, a_dst1,
           We1, a_e1, b1, W2, a_src2, a_dst2, We2, a_e2, b2, empty_emb,
           Rw1, Rb1, Rw2, Rb2):
    pad_e = E_PAD - E
    src3 = jnp.concatenate(
        [edge_index[0], jnp.zeros((pad_e,), jnp.int32)]).reshape(NW, CB, CW)
    dst_fill = N + jnp.arange(pad_e, dtype=jnp.int32) % (NP_ - N)
    dst3 = jnp.concatenate(
        [edge_index[1], dst_fill]).reshape(NW, CB, CW)

    aev = jnp.stack([a_e1, a_e2], axis=0)
    av1 = jnp.stack([a_src1, a_dst1], axis=1)
    av2 = jnp.stack([a_src2, a_dst2], axis=1)

    x_pad = jnp.concatenate(
        [x, jnp.zeros((NP_ - N, D), jnp.float32)], axis=0)
    batch_pad = jnp.concatenate(
        [batch, jnp.full((NP_ - N,), -1, batch.dtype)])

    ae = _edge_kernel(edge_attr, We1, We2, aev)
    ae_p = jnp.concatenate([ae, jnp.zeros((pad_e, 2), jnp.float32)], axis=0)
    ae1 = ae_p[:, 0].reshape(NW, CB, CW)
    ae2 = ae_p[:, 1].reshape(NW, CB, CW)

    h1, sad1 = _node_kernel(x_pad, W1, av1)
    ex1, den1 = _sc_alpha_pass(src3, dst3, ae1, sad1[:, 0], sad1[:, 1])
    acc1 = _sc_scatter_pass(h1, src3, dst3, ex1)
    h2, sad2 = _mid_kernel(acc1, den1, b1.reshape(1, H), W2, av2)
    ex2, den2 = _sc_alpha_pass(src3, dst3, ae2, sad2[:, 0], sad2[:, 1])
    acc2 = _sc_scatter_pass(h2, src3, dst3, ex2)
    out = _final_kernel(acc2, den2, batch_pad.reshape(NP_ // NB, NB),
                        b2.reshape(1, H), empty_emb.reshape(1, H),
                        Rw1, Rb1.reshape(1, RH), Rw2, Rb2.reshape(1, 1))
    return out[:, 0]


# speedup vs baseline: 13.0858x; 1.0988x over previous
import dataclasses
import functools

import jax
import jax.numpy as jnp
from jax import lax
from jax.experimental import pallas as pl
from jax.experimental.pallas import tpu as pltpu
from jax.experimental.pallas import tpu_sc as plsc

N = 10000
E = 320000
D = 128
H = 128
G = 64
RH = 256

NC = 2
NS = 16
NW = NC * NS
CW = 128
CB = 80
EPT_P = CB * CW
E_PAD = NW * EPT_P
SBC = 16
NP_ = 10240
NB = 1024
EB = 8000
RPTP = NP_ // NS


def _dot(a, b):
    return jnp.dot(a, b, preferred_element_type=jnp.float32)


def _proj(h, av):
    s0 = jnp.sum(h * av[:, 0][None, :], axis=1, keepdims=True)
    s1 = jnp.sum(h * av[:, 1][None, :], axis=1, keepdims=True)
    return jnp.concatenate([s0, s1], axis=1)


def _node_body(x_ref, w_ref, av_ref, h_ref, sad_ref):
    h = _dot(x_ref[...], w_ref[...])
    h_ref[...] = h
    sad_ref[...] = _proj(h, av_ref[...])


def _node_kernel(x, W, av):
    return pl.pallas_call(
        _node_body,
        grid=(NP_ // NB,),
        in_specs=[
            pl.BlockSpec((NB, D), lambda i: (i, 0)),
            pl.BlockSpec((D, H), lambda i: (0, 0)),
            pl.BlockSpec((H, 2), lambda i: (0, 0)),
        ],
        out_specs=[
            pl.BlockSpec((NB, H), lambda i: (i, 0)),
            pl.BlockSpec((NB, 2), lambda i: (i, 0)),
        ],
        out_shape=[
            jax.ShapeDtypeStruct((NP_, H), jnp.float32),
            jax.ShapeDtypeStruct((NP_, 2), jnp.float32),
        ],
    )(x, W, av)


def _edge_body(ea_ref, we1_ref, we2_ref, aev_ref, o_ref):
    ea = ea_ref[...]
    aev = aev_ref[...]
    e1 = _dot(ea, we1_ref[...])
    e2 = _dot(ea, we2_ref[...])
    o1 = jnp.sum(e1 * aev[0][None, :], axis=1, keepdims=True)
    o2 = jnp.sum(e2 * aev[1][None, :], axis=1, keepdims=True)
    o_ref[...] = jnp.concatenate([o1, o2], axis=1)


def _edge_kernel(edge_attr, We1, We2, aev):
    return pl.pallas_call(
        _edge_body,
        grid=(E // EB,),
        in_specs=[
            pl.BlockSpec((EB, 16), lambda i: (i, 0)),
            pl.BlockSpec((16, H), lambda i: (0, 0)),
            pl.BlockSpec((16, H), lambda i: (0, 0)),
            pl.BlockSpec((2, H), lambda i: (0, 0)),
        ],
        out_specs=pl.BlockSpec((EB, 2), lambda i: (i, 0)),
        out_shape=jax.ShapeDtypeStruct((E, 2), jnp.float32),
    )(edge_attr, We1, We2, aev)


def _mid_body(acc_ref, den_ref, b_ref, w_ref, av_ref, h2_ref, sad_ref):
    i = pl.program_id(0)
    acc = acc_ref[0] + acc_ref[1]
    den = jnp.sum(den_ref[:, pl.ds(i * NB, NB)], axis=0)
    h1 = jnp.maximum(acc / (den[:, None] + 1e-16) + b_ref[...], 0.0)
    h2 = _dot(h1, w_ref[...])
    h2_ref[...] = h2
    sad_ref[...] = _proj(h2, av_ref[...])


def _mid_kernel(acc, den, b, W, av):
    return pl.pallas_call(
        _mid_body,
        grid=(NP_ // NB,),
        in_specs=[
            pl.BlockSpec((NC, NB, H), lambda i: (0, i, 0)),
            pl.BlockSpec((NW, NP_), lambda i: (0, 0)),
            pl.BlockSpec((1, H), lambda i: (0, 0)),
            pl.BlockSpec((H, H), lambda i: (0, 0)),
            pl.BlockSpec((H, 2), lambda i: (0, 0)),
        ],
        out_specs=[
            pl.BlockSpec((NB, H), lambda i: (i, 0)),
            pl.BlockSpec((NB, 2), lambda i: (i, 0)),
        ],
        out_shape=[
            jax.ShapeDtypeStruct((NP_, H), jnp.float32),
            jax.ShapeDtypeStruct((NP_, 2), jnp.float32),
        ],
    )(acc, den, b, W, av)


def _final_body(acc_ref, den_ref, batch_ref, b2_ref, emb_ref, rw1_ref,
                rb1_ref, rw2_ref, rb2_ref, o_ref):
    pooled = jnp.zeros((G, H), jnp.float32)
    for blk in range(NP_ // NB):
        sl = pl.ds(blk * NB, NB)
        acc = acc_ref[0, sl, :] + acc_ref[1, sl, :]
        den = jnp.sum(den_ref[:, sl], axis=0)[:, None]
        h2 = acc / (den + 1e-16) + b2_ref[...]
        brow = batch_ref[blk, :][None, :]
        oh = (lax.broadcasted_iota(jnp.int32, (G, NB), 0) == brow)
        pooled = pooled + jnp.dot(
            oh.astype(jnp.float32), h2,
            preferred_element_type=jnp.float32,
            precision=lax.Precision.HIGHEST)
    mask = jnp.sum(pooled, axis=1, keepdims=True) == 0.0
    pooled = jnp.where(mask, emb_ref[...], pooled)
    hid = jnp.maximum(_dot(pooled, rw1_ref[...]) + rb1_ref[...], 0.0)
    o_ref[...] = _dot(hid, rw2_ref[...]) + rb2_ref[...]


def _final_kernel(acc, den, batch2, b2, emb, Rw1, Rb1, Rw2, Rb2):
    return pl.pallas_call(
        _final_body,
        out_shape=jax.ShapeDtypeStruct((G, 1), jnp.float32),
    )(acc, den, batch2, b2, emb, Rw1, Rb1, Rw2, Rb2)


def _sc_compiler_params():
    cp = pltpu.CompilerParams()
    if "needs_layout_passes" in pltpu.CompilerParams.__dataclass_fields__:
        cp = dataclasses.replace(cp, needs_layout_passes=False)
    return cp


def _sc_alpha_pass(src3, dst3, ae3, asrc, adst):
    mesh = plsc.VectorSubcoreMesh(core_axis_name="c", subcore_axis_name="s")

    @functools.partial(
        pl.kernel,
        mesh=mesh,
        compiler_params=_sc_compiler_params(),
        out_type=[
            jax.ShapeDtypeStruct((NW, CB, CW), jnp.float32),
            jax.ShapeDtypeStruct((NW, NP_), jnp.float32),
        ],
        scratch_types=[
            pltpu.VMEM((NP_,), jnp.float32),
            pltpu.VMEM((NP_,), jnp.float32),
            pltpu.VMEM((NP_,), jnp.float32),
            pltpu.VMEM((CB, CW), jnp.int32),
            pltpu.VMEM((CB, CW), jnp.int32),
            pltpu.VMEM((CB, CW), jnp.float32),
            pltpu.VMEM((CB, CW), jnp.float32),
            pltpu.VMEM((16,), jnp.int32),
            pltpu.VMEM((16,), jnp.float32),
        ],
    )
    def k(src_hbm, dst_hbm, ae_hbm, asrc_hbm, adst_hbm, ex_hbm, den_hbm,
          asrc_v, adst_v, den_v, src_v, dst_v, ae_v, ex_v, kbuf, sbuf):
        cid = lax.axis_index("c")
        sid = lax.axis_index("s")
        wid = sid * NC + cid
        iota16 = lax.iota(jnp.int32, 16)

        pltpu.sync_copy(src_hbm.at[wid], src_v)
        pltpu.sync_copy(dst_hbm.at[wid], dst_v)
        pltpu.sync_copy(ae_hbm.at[wid], ae_v)
        pltpu.sync_copy(asrc_hbm, asrc_v)
        pltpu.sync_copy(adst_hbm, adst_v)

        @pl.loop(0, NP_, step=16)
        def _(i):
            den_v[pl.ds(i, 16)] = jnp.zeros((16,), jnp.float32)

        @pl.loop(0, CB)
        def _(r):
            @pl.loop(0, CW, step=16)
            def _(j):
                s16 = src_v[r, pl.ds(j, 16)]
                d16 = dst_v[r, pl.ds(j, 16)]
                a = (plsc.load_gather(asrc_v, [s16])
                     + plsc.load_gather(adst_v, [d16])
                     + ae_v[r, pl.ds(j, 16)])
                a = jnp.where(a >= 0.0, a, 0.2 * a)
                exv = jnp.exp(a)
                ex_v[r, pl.ds(j, 16)] = exv

                ks, vs = plsc.sort_key_val(d16, exv)
                kbuf[...] = ks
                run = vs
                for dlt in (1, 2, 4, 8):
                    sbuf[...] = run
                    idx = jnp.maximum(iota16 - dlt, 0)
                    kd = plsc.load_gather(kbuf, [idx])
                    sd = plsc.load_gather(sbuf, [idx])
                    take = (kd == ks) & (iota16 >= dlt)
                    run = run + jnp.where(take, sd, 0.0)
                knext = plsc.load_gather(kbuf, [jnp.minimum(iota16 + 1, 15)])
                last = (knext != ks) | (iota16 == 15)
                plsc.addupdate_scatter(den_v, [ks], run, mask=last)

        pltpu.sync_copy(ex_v, ex_hbm.at[wid])
        pltpu.sync_copy(den_v, den_hbm.at[wid])

    return k(src3, dst3, ae3, asrc, adst)


def _sc_scatter_pass(h, src3, dst3, ex3):
    mesh = plsc.VectorSubcoreMesh(core_axis_name="c", subcore_axis_name="s")

    @functools.partial(
        pl.kernel,
        mesh=mesh,
        compiler_params=_sc_compiler_params(),
        out_type=jax.ShapeDtypeStruct((NC, NP_, H), jnp.float32),
        scratch_types=[
            pltpu.VMEM((SBC, CW), jnp.int32),
            pltpu.VMEM((SBC, CW), jnp.int32),
            pltpu.VMEM((SBC, CW), jnp.float32),
            pltpu.VMEM((CW, H), jnp.float32),
            pltpu.VMEM((CW, H), jnp.float32),
            pltpu.VMEM_SHARED((NP_, H), jnp.float32),
            pltpu.SemaphoreType.DMA,
            pltpu.SemaphoreType.DMA,
            pltpu.SemaphoreType.DMA,
            pltpu.SemaphoreType.DMA,
        ],
    )
    def k(h_hbm, src_hbm, dst_hbm, ex_hbm, acc_hbm,
          sblk, dblk, eblk, rows_a, rows_b, acc_sh,
          sem_ga, sem_gb, sem_sa, sem_sb):
        cid = lax.axis_index("c")
        sid = lax.axis_index("s")
        wid = sid * NC + cid

        def scale_rows(rows, c_loc):
            @pl.loop(0, CW, step=16)
            def _(j):
                exv = eblk[c_loc, pl.ds(j, 16)]
                for t in range(16):
                    scv = jnp.full((16,), exv[t], jnp.float32)
                    for v in range(H // 16):
                        rows[j + t, pl.ds(v * 16, 16)] = (
                            rows[j + t, pl.ds(v * 16, 16)] * scv)

        def g_issue(rows, c_loc, sem):
            pltpu.async_copy(h_hbm.at[sblk.at[c_loc]], rows, sem)

        def g_wait(rows, c_loc, sem):
            pltpu.make_async_copy(h_hbm.at[sblk.at[c_loc]], rows, sem).wait()

        def s_issue(rows, c_loc, sem):
            pltpu.async_copy(rows, acc_sh.at[dblk.at[c_loc]], sem, add=True)

        def s_wait(rows, c_loc, sem):
            pltpu.make_async_copy(rows, acc_sh.at[dblk.at[c_loc]],
                                  sem).wait()

        @pl.loop(0, CW)
        def _(r):
            @pl.loop(0, H, step=16)
            def _(c2):
                rows_a[r, pl.ds(c2, 16)] = jnp.zeros((16,), jnp.float32)

        zbase = RPTP * sid
        for j in range(RPTP // CW):
            pltpu.sync_copy(rows_a, acc_sh.at[pl.ds(zbase + j * CW, CW)])

        plsc.subcore_barrier()

        for sb in range(CB // SBC):
            pltpu.sync_copy(src_hbm.at[wid, pl.ds(sb * SBC, SBC)], sblk)
            pltpu.sync_copy(dst_hbm.at[wid, pl.ds(sb * SBC, SBC)], dblk)
            pltpu.sync_copy(ex_hbm.at[wid, pl.ds(sb * SBC, SBC)], eblk)

            g_issue(rows_a, 0, sem_ga)
            g_issue(rows_b, 1, sem_gb)

            @pl.loop(0, SBC // 2)
            def _(p):
                ca = 2 * p
                cb = 2 * p + 1
                g_wait(rows_a, ca, sem_ga)
                scale_rows(rows_a, ca)
                s_issue(rows_a, ca, sem_sa)
                g_wait(rows_b, cb, sem_gb)
                scale_rows(rows_b, cb)
                s_issue(rows_b, cb, sem_sb)
                s_wait(rows_a, ca, sem_sa)

                @pl.when(p < SBC // 2 - 1)
                def _():
                    g_issue(rows_a, ca + 2, sem_ga)
                s_wait(rows_b, cb, sem_sb)

                @pl.when(p < SBC // 2 - 1)
                def _():
                    g_issue(rows_b, cb + 2, sem_gb)

        plsc.subcore_barrier()

        for j in range(RPTP // CW):
            pltpu.sync_copy(acc_sh.at[pl.ds(zbase + j * CW, CW)],
                            acc_hbm.at[cid, pl.ds(zbase + j * CW, CW)])

    return k(h, src3, dst3, ex3)


def kernel(x, edge_index, edge_attr, batch, batch_size, W1, a_src1, a_dst1,
           We1, a_e1, b1, W2, a_src2, a_dst2, We2, a_e2, b2, empty_emb,
           Rw1, Rb1, Rw2, Rb2):
    pad_e = E_PAD - E
    src3 = jnp.concatenate(
        [edge_index[0], jnp.zeros((pad_e,), jnp.int32)]).reshape(NW, CB, CW)
    dst_fill = N + jnp.arange(pad_e, dtype=jnp.int32) % (NP_ - N)
    dst3 = jnp.concatenate(
        [edge_index[1], dst_fill]).reshape(NW, CB, CW)

    aev = jnp.stack([a_e1, a_e2], axis=0)
    av1 = jnp.stack([a_src1, a_dst1], axis=1)
    av2 = jnp.stack([a_src2, a_dst2], axis=1)

    x_pad = jnp.concatenate(
        [x, jnp.zeros((NP_ - N, D), jnp.float32)], axis=0)
    batch_pad = jnp.concatenate(
        [batch, jnp.full((NP_ - N,), -1, batch.dtype)])

    ae = _edge_kernel(edge_attr, We1, We2, aev)
    ae_p = jnp.concatenate([ae, jnp.zeros((pad_e, 2), jnp.float32)], axis=0)
    ae1 = ae_p[:, 0].reshape(NW, CB, CW)
    ae2 = ae_p[:, 1].reshape(NW, CB, CW)

    h1, sad1 = _node_kernel(x_pad, W1, av1)
    ex1, den1 = _sc_alpha_pass(src3, dst3, ae1, sad1[:, 0], sad1[:, 1])
    acc1 = _sc_scatter_pass(h1, src3, dst3, ex1)
    h2, sad2 = _mid_kernel(acc1, den1, b1.reshape(1, H), W2, av2)
    ex2, den2 = _sc_alpha_pass(src3, dst3, ae2, sad2[:, 0], sad2[:, 1])
    acc2 = _sc_scatter_pass(h2, src3, dst3, ex2)
    out = _final_kernel(acc2, den2, batch_pad.reshape(NP_ // NB, NB),
                        b2.reshape(1, H), empty_emb.reshape(1, H),
                        Rw1, Rb1.reshape(1, RH), Rw2, Rb2.reshape(1, 1))
    return out[:, 0]


# speedup vs baseline: 13.4670x; 1.0291x over previous
import dataclasses
import functools

import jax
import jax.numpy as jnp
from jax import lax
from jax.experimental import pallas as pl
from jax.experimental.pallas import tpu as pltpu
from jax.experimental.pallas import tpu_sc as plsc

N = 10000
E = 320000
D = 128
H = 128
G = 64
RH = 256

NC = 2
NS = 16
NW = NC * NS
CW = 128
CB = 80
EPT_P = CB * CW
E_PAD = NW * EPT_P
SBC = 16
NP_ = 10240
NB = 1024
EB = 8000
RPTP = NP_ // NS


def _dot(a, b):
    return jnp.dot(a, b, preferred_element_type=jnp.float32)


def _proj(h, av):
    s0 = jnp.sum(h * av[:, 0][None, :], axis=1, keepdims=True)
    s1 = jnp.sum(h * av[:, 1][None, :], axis=1, keepdims=True)
    return jnp.concatenate([s0, s1], axis=1)


def _node_body(x_ref, w_ref, av_ref, h_ref, sad_ref):
    h = _dot(x_ref[...], w_ref[...])
    h_ref[...] = h
    sad_ref[...] = _proj(h, av_ref[...])


def _node_kernel(x, W, av):
    return pl.pallas_call(
        _node_body,
        grid=(NP_ // NB,),
        in_specs=[
            pl.BlockSpec((NB, D), lambda i: (i, 0)),
            pl.BlockSpec((D, H), lambda i: (0, 0)),
            pl.BlockSpec((H, 2), lambda i: (0, 0)),
        ],
        out_specs=[
            pl.BlockSpec((NB, H), lambda i: (i, 0)),
            pl.BlockSpec((NB, 2), lambda i: (i, 0)),
        ],
        out_shape=[
            jax.ShapeDtypeStruct((NP_, H), jnp.float32),
            jax.ShapeDtypeStruct((NP_, 2), jnp.float32),
        ],
    )(x, W, av)


def _edge_body(ea_ref, we1_ref, we2_ref, aev_ref, o_ref):
    ea = ea_ref[...]
    aev = aev_ref[...]
    e1 = _dot(ea, we1_ref[...])
    e2 = _dot(ea, we2_ref[...])
    o1 = jnp.sum(e1 * aev[0][None, :], axis=1, keepdims=True)
    o2 = jnp.sum(e2 * aev[1][None, :], axis=1, keepdims=True)
    o_ref[...] = jnp.concatenate([o1, o2], axis=1)


def _edge_kernel(edge_attr, We1, We2, aev):
    return pl.pallas_call(
        _edge_body,
        grid=(E // EB,),
        in_specs=[
            pl.BlockSpec((EB, 16), lambda i: (i, 0)),
            pl.BlockSpec((16, H), lambda i: (0, 0)),
            pl.BlockSpec((16, H), lambda i: (0, 0)),
            pl.BlockSpec((2, H), lambda i: (0, 0)),
        ],
        out_specs=pl.BlockSpec((EB, 2), lambda i: (i, 0)),
        out_shape=jax.ShapeDtypeStruct((E, 2), jnp.float32),
    )(edge_attr, We1, We2, aev)


def _mid_body(acc_ref, den_ref, b_ref, w_ref, av_ref, h2_ref, sad_ref):
    i = pl.program_id(0)
    acc = acc_ref[0] + acc_ref[1]
    den = jnp.sum(den_ref[:, pl.ds(i * NB, NB)], axis=0)
    h1 = jnp.maximum(acc / (den[:, None] + 1e-16) + b_ref[...], 0.0)
    h2 = _dot(h1, w_ref[...])
    h2_ref[...] = h2
    sad_ref[...] = _proj(h2, av_ref[...])


def _mid_kernel(acc, den, b, W, av):
    return pl.pallas_call(
        _mid_body,
        grid=(NP_ // NB,),
        in_specs=[
            pl.BlockSpec((NC, NB, H), lambda i: (0, i, 0)),
            pl.BlockSpec((NW, NP_), lambda i: (0, 0)),
            pl.BlockSpec((1, H), lambda i: (0, 0)),
            pl.BlockSpec((H, H), lambda i: (0, 0)),
            pl.BlockSpec((H, 2), lambda i: (0, 0)),
        ],
        out_specs=[
            pl.BlockSpec((NB, H), lambda i: (i, 0)),
            pl.BlockSpec((NB, 2), lambda i: (i, 0)),
        ],
        out_shape=[
            jax.ShapeDtypeStruct((NP_, H), jnp.float32),
            jax.ShapeDtypeStruct((NP_, 2), jnp.float32),
        ],
    )(acc, den, b, W, av)


def _final_body(acc_ref, den_ref, batch_ref, b2_ref, emb_ref, rw1_ref,
                rb1_ref, rw2_ref, rb2_ref, o_ref):
    pooled = jnp.zeros((G, H), jnp.float32)
    for blk in range(NP_ // NB):
        sl = pl.ds(blk * NB, NB)
        acc = acc_ref[0, sl, :] + acc_ref[1, sl, :]
        den = jnp.sum(den_ref[:, sl], axis=0)[:, None]
        h2 = acc / (den + 1e-16) + b2_ref[...]
        brow = batch_ref[blk, :][None, :]
        oh = (lax.broadcasted_iota(jnp.int32, (G, NB), 0) == brow)
        pooled = pooled + jnp.dot(
            oh.astype(jnp.float32), h2,
            preferred_element_type=jnp.float32,
            precision=lax.Precision.HIGHEST)
    mask = jnp.sum(pooled, axis=1, keepdims=True) == 0.0
    pooled = jnp.where(mask, emb_ref[...], pooled)
    hid = jnp.maximum(_dot(pooled, rw1_ref[...]) + rb1_ref[...], 0.0)
    o_ref[...] = _dot(hid, rw2_ref[...]) + rb2_ref[...]


def _final_kernel(acc, den, batch2, b2, emb, Rw1, Rb1, Rw2, Rb2):
    return pl.pallas_call(
        _final_body,
        out_shape=jax.ShapeDtypeStruct((G, 1), jnp.float32),
    )(acc, den, batch2, b2, emb, Rw1, Rb1, Rw2, Rb2)


def _sc_compiler_params():
    cp = pltpu.CompilerParams()
    if "needs_layout_passes" in pltpu.CompilerParams.__dataclass_fields__:
        cp = dataclasses.replace(cp, needs_layout_passes=False)
    return cp


def _sc_alpha_pass(src3, dst3, ae3, asrc, adst):
    mesh = plsc.VectorSubcoreMesh(core_axis_name="c", subcore_axis_name="s")

    @functools.partial(
        pl.kernel,
        mesh=mesh,
        compiler_params=_sc_compiler_params(),
        out_type=[
            jax.ShapeDtypeStruct((NW, CB, CW), jnp.float32),
            jax.ShapeDtypeStruct((NW, NP_), jnp.float32),
        ],
        scratch_types=[
            pltpu.VMEM((NP_,), jnp.float32),
            pltpu.VMEM((NP_,), jnp.float32),
            pltpu.VMEM((NP_,), jnp.float32),
            pltpu.VMEM((CB, CW), jnp.int32),
            pltpu.VMEM((CB, CW), jnp.int32),
            pltpu.VMEM((CB, CW), jnp.float32),
            pltpu.VMEM((CB, CW), jnp.float32),
            pltpu.VMEM((16,), jnp.int32),
            pltpu.VMEM((16,), jnp.float32),
        ],
    )
    def k(src_hbm, dst_hbm, ae_hbm, asrc_hbm, adst_hbm, ex_hbm, den_hbm,
          asrc_v, adst_v, den_v, src_v, dst_v, ae_v, ex_v, kbuf, sbuf):
        cid = lax.axis_index("c")
        sid = lax.axis_index("s")
        wid = sid * NC + cid
        iota16 = lax.iota(jnp.int32, 16)

        pltpu.sync_copy(src_hbm.at[wid], src_v)
        pltpu.sync_copy(dst_hbm.at[wid], dst_v)
        pltpu.sync_copy(ae_hbm.at[wid], ae_v)
        pltpu.sync_copy(asrc_hbm, asrc_v)
        pltpu.sync_copy(adst_hbm, adst_v)

        @pl.loop(0, NP_, step=16)
        def _(i):
            den_v[pl.ds(i, 16)] = jnp.zeros((16,), jnp.float32)

        @pl.loop(0, CB)
        def _(r):
            @pl.loop(0, CW, step=16)
            def _(j):
                s16 = src_v[r, pl.ds(j, 16)]
                d16 = dst_v[r, pl.ds(j, 16)]
                a = (plsc.load_gather(asrc_v, [s16])
                     + plsc.load_gather(adst_v, [d16])
                     + ae_v[r, pl.ds(j, 16)])
                a = jnp.where(a >= 0.0, a, 0.2 * a)
                exv = jnp.exp(a)
                ex_v[r, pl.ds(j, 16)] = exv

                ks, vs = plsc.sort_key_val(d16, exv)
                kbuf[...] = ks
                run = vs
                for dlt in (1, 2, 4, 8):
                    sbuf[...] = run
                    idx = jnp.maximum(iota16 - dlt, 0)
                    kd = plsc.load_gather(kbuf, [idx])
                    sd = plsc.load_gather(sbuf, [idx])
                    take = (kd == ks) & (iota16 >= dlt)
                    run = run + jnp.where(take, sd, 0.0)
                knext = plsc.load_gather(kbuf, [jnp.minimum(iota16 + 1, 15)])
                last = (knext != ks) | (iota16 == 15)
                plsc.addupdate_scatter(den_v, [ks], run, mask=last)

        pltpu.sync_copy(ex_v, ex_hbm.at[wid])
        pltpu.sync_copy(den_v, den_hbm.at[wid])

    return k(src3, dst3, ae3, asrc, adst)


def _sc_scatter_pass(h, src3, dst3, ex3):
    mesh = plsc.VectorSubcoreMesh(core_axis_name="c", subcore_axis_name="s")

    @functools.partial(
        pl.kernel,
        mesh=mesh,
        compiler_params=_sc_compiler_params(),
        out_type=jax.ShapeDtypeStruct((NC, NP_, H), jnp.float32),
        scratch_types=[
            pltpu.VMEM((SBC, CW), jnp.int32),
            pltpu.VMEM((SBC, CW), jnp.int32),
            pltpu.VMEM((SBC, CW), jnp.float32),
            pltpu.VMEM((CW, H), jnp.float32),
            pltpu.VMEM((CW, H), jnp.float32),
            pltpu.VMEM_SHARED((NP_, H), jnp.float32),
            pltpu.SemaphoreType.DMA,
            pltpu.SemaphoreType.DMA,
            pltpu.SemaphoreType.DMA,
            pltpu.SemaphoreType.DMA,
        ],
    )
    def k(h_hbm, src_hbm, dst_hbm, ex_hbm, acc_hbm,
          sblk, dblk, eblk, rows_a, rows_b, acc_sh,
          sem_ga, sem_gb, sem_sa, sem_sb):
        cid = lax.axis_index("c")
        sid = lax.axis_index("s")
        wid = sid * NC + cid

        def scale_rows(rows, c_loc):
            @pl.loop(0, CW, step=16)
            def _(j):
                exv = eblk[c_loc, pl.ds(j, 16)]
                for t in range(16):
                    scv = jnp.full((16,), exv[t], jnp.float32)
                    for v in range(H // 16):
                        rows[j + t, pl.ds(v * 16, 16)] = (
                            rows[j + t, pl.ds(v * 16, 16)] * scv)

        def g_issue(rows, c_loc, sem):
            pltpu.async_copy(h_hbm.at[sblk.at[c_loc]], rows, sem)

        def g_wait(rows, c_loc, sem):
            pltpu.make_async_copy(h_hbm.at[sblk.at[c_loc]], rows, sem).wait()

        def s_issue(rows, c_loc, sem):
            pltpu.async_copy(rows, acc_sh.at[dblk.at[c_loc]], sem, add=True)

        def s_wait(rows, c_loc, sem):
            pltpu.make_async_copy(rows, acc_sh.at[dblk.at[c_loc]],
                                  sem).wait()

        @pl.loop(0, CW)
        def _(r):
            @pl.loop(0, H, step=16)
            def _(c2):
                rows_a[r, pl.ds(c2, 16)] = jnp.zeros((16,), jnp.float32)

        zbase = RPTP * sid
        for j in range(RPTP // CW):
            pltpu.sync_copy(rows_a, acc_sh.at[pl.ds(zbase + j * CW, CW)])

        plsc.subcore_barrier()

        for sb in range(CB // SBC):
            pltpu.sync_copy(src_hbm.at[wid, pl.ds(sb * SBC, SBC)], sblk)
            pltpu.sync_copy(dst_hbm.at[wid, pl.ds(sb * SBC, SBC)], dblk)
            pltpu.sync_copy(ex_hbm.at[wid, pl.ds(sb * SBC, SBC)], eblk)

            g_issue(rows_a, 0, sem_ga)
            g_issue(rows_b, 1, sem_gb)

            @pl.loop(0, SBC // 2)
            def _(p):
                ca = 2 * p
                cb = 2 * p + 1
                g_wait(rows_a, ca, sem_ga)
                scale_rows(rows_a, ca)
                s_issue(rows_a, ca, sem_sa)
                s_wait(rows_a, ca, sem_sa)

                @pl.when(p < SBC // 2 - 1)
                def _():
                    g_issue(rows_a, ca + 2, sem_ga)
                g_wait(rows_b, cb, sem_gb)
                scale_rows(rows_b, cb)
                s_issue(rows_b, cb, sem_sb)
                s_wait(rows_b, cb, sem_sb)

                @pl.when(p < SBC // 2 - 1)
                def _():
                    g_issue(rows_b, cb + 2, sem_gb)

        plsc.subcore_barrier()

        pltpu.sync_copy(acc_sh.at[pl.ds(zbase, RPTP)],
                        acc_hbm.at[cid, pl.ds(zbase, RPTP)])

    return k(h, src3, dst3, ex3)


def kernel(x, edge_index, edge_attr, batch, batch_size, W1, a_src1, a_dst1,
           We1, a_e1, b1, W2, a_src2, a_dst2, We2, a_e2, b2, empty_emb,
           Rw1, Rb1, Rw2, Rb2):
    pad_e = E_PAD - E
    src3 = jnp.concatenate(
        [edge_index[0], jnp.zeros((pad_e,), jnp.int32)]).reshape(NW, CB, CW)
    dst_fill = N + jnp.arange(pad_e, dtype=jnp.int32) % (NP_ - N)
    dst3 = jnp.concatenate(
        [edge_index[1], dst_fill]).reshape(NW, CB, CW)

    aev = jnp.stack([a_e1, a_e2], axis=0)
    av1 = jnp.stack([a_src1, a_dst1], axis=1)
    av2 = jnp.stack([a_src2, a_dst2], axis=1)

    x_pad = jnp.concatenate(
        [x, jnp.zeros((NP_ - N, D), jnp.float32)], axis=0)
    batch_pad = jnp.concatenate(
        [batch, jnp.full((NP_ - N,), -1, batch.dtype)])

    ae = _edge_kernel(edge_attr, We1, We2, aev)
    ae_p = jnp.concatenate([ae, jnp.zeros((pad_e, 2), jnp.float32)], axis=0)
    ae1 = ae_p[:, 0].reshape(NW, CB, CW)
    ae2 = ae_p[:, 1].reshape(NW, CB, CW)

    h1, sad1 = _node_kernel(x_pad, W1, av1)
    ex1, den1 = _sc_alpha_pass(src3, dst3, ae1, sad1[:, 0], sad1[:, 1])
    acc1 = _sc_scatter_pass(h1, src3, dst3, ex1)
    h2, sad2 = _mid_kernel(acc1, den1, b1.reshape(1, H), W2, av2)
    ex2, den2 = _sc_alpha_pass(src3, dst3, ae2, sad2[:, 0], sad2[:, 1])
    acc2 = _sc_scatter_pass(h2, src3, dst3, ex2)
    out = _final_kernel(acc2, den2, batch_pad.reshape(NP_ // NB, NB),
                        b2.reshape(1, H), empty_emb.reshape(1, H),
                        Rw1, Rb1.reshape(1, RH), Rw2, Rb2.reshape(1, 1))
    return out[:, 0]
